# baseline (speedup 1.0000x reference)
import functools

import jax
import jax.numpy as jnp
import numpy as np
from jax import lax
from jax.experimental import pallas as pl
from jax.experimental.pallas import tpu as pltpu
from jax.experimental.pallas import tpu_sc as plsc

N = 10000
E = 320000
D = 128
DH = D // 2
DP = DH // 2
NC = 2
NS = 16
EPT = E // NS
CH = 80
NCHUNK = EPT // CH
DEPTH = 6
NPROC = 252
NPADC = NPROC + DEPTH
RPT = 624
RTAIL = N - NS * RPT

_PERM = np.concatenate([
    np.stack([np.arange(0, 16), np.arange(32, 48)], 1).ravel(),
    np.stack([np.arange(16, 32), np.arange(48, 64)], 1).ravel(),
])

_mesh = plsc.VectorSubcoreMesh(core_axis_name="c", subcore_axis_name="s")


@functools.partial(
    pl.kernel,
    mesh=_mesh,
    compiler_params=pltpu.CompilerParams(
        use_tc_tiling_on_sc=False, needs_layout_passes=False),
    out_type=jax.ShapeDtypeStruct((NC, N, DH), jnp.float32),
    scratch_types=(
        [pltpu.VMEM((NPADC, CH), jnp.int32)]
        + [pltpu.VMEM((CH,), jnp.int32) for _ in range(DEPTH)]
        + [pltpu.VMEM((CH,), jnp.int32) for _ in range(DEPTH)]
        + [pltpu.VMEM((CH, DP), jnp.int32) for _ in range(DEPTH)]
        + [pltpu.VMEM((CH, DH), jnp.float32) for _ in range(DEPTH)]
        + [pltpu.VMEM((CH,), jnp.float32) for _ in range(DEPTH)]
        + [pltpu.VMEM_SHARED((N, DH), jnp.float32)]
        + [pltpu.SemaphoreType.DMA] * (3 * DEPTH)
    ),
)
def _aggregate(xs_hbm, sd_hbm, ew_hbm, out_hbm, *refs):
    sd_v = refs[0]
    idxb = refs[1:1 + DEPTH]
    dstb = refs[1 + DEPTH:1 + 2 * DEPTH]
    gb = refs[1 + 2 * DEPTH:1 + 3 * DEPTH]
    sb = refs[1 + 3 * DEPTH:1 + 4 * DEPTH]
    ewb = refs[1 + 4 * DEPTH:1 + 5 * DEPTH]
    acc = refs[1 + 5 * DEPTH]
    gsem = refs[2 + 5 * DEPTH:2 + 6 * DEPTH]
    ssem = refs[2 + 6 * DEPTH:2 + 7 * DEPTH]
    wsem = refs[2 + 7 * DEPTH:2 + 8 * DEPTH]

    cc = lax.axis_index("c")
    ss = lax.axis_index("s")
    ccn = cc * N

    pltpu.sync_copy(sd_hbm.at[ss], sd_v)

    lomask = jnp.int32((1 << 15) - 1)

    def _mk_src(c, buf):
        for j in range(CH // 16):
            sl = pl.ds(j * 16, 16)
            buf[sl] = (sd_v[c, sl] & lomask) + ccn

    def _mk_dst(c, buf):
        for j in range(CH // 16):
            sl = pl.ds(j * 16, 16)
            buf[sl] = sd_v[c, sl] >> 15

    def _zrow(i, _):
        for b in range(DEPTH):
            for j in range(DH // 16):
                sb[b][i, pl.ds(j * 16, 16)] = jnp.zeros((16,), jnp.float32)
        return 0
    lax.fori_loop(0, CH, _zrow, 0)
    for b in range(DEPTH):
        for j in range(CH // 16):
            dstb[b][pl.ds(j * 16, 16)] = jnp.zeros((16,), jnp.int32)

    for k in range(7):
        pltpu.sync_copy(sb[0], acc.at[pl.ds(ss * RPT + k * CH, CH)])
    pltpu.sync_copy(sb[0].at[pl.ds(0, RPT - 7 * CH)],
                    acc.at[pl.ds(ss * RPT + 7 * CH, RPT - 7 * CH)])

    @pl.when(ss == NS - 1)
    def _zero_tail():
        pltpu.sync_copy(sb[0].at[pl.ds(0, RTAIL)],
                        acc.at[pl.ds(NS * RPT, RTAIL)])

    plsc.subcore_barrier()

    himask = jnp.int32(-65536)

    def _scale(gbuf, sbuf, ewbuf):
        def _grp(g, _):
            wvec = ewbuf[pl.ds(g * 16, 16)]
            for l in range(16):
                e = g * 16 + l
                w = wvec[l]
                for h in range(2):
                    v = gbuf[e, pl.ds(h * 16, 16)]
                    lo = plsc.bitcast(v << 16, jnp.float32)
                    hi = plsc.bitcast(v & himask, jnp.float32)
                    sbuf[e, pl.ds(h * 16, 16)] = lo * w
                    sbuf[e, pl.ds(h * 16 + 32, 16)] = hi * w
            return 0
        lax.fori_loop(0, CH // 16, _grp, 0)

    for b in range(DEPTH):
        _mk_src(b, idxb[b])
        pltpu.async_copy(xs_hbm.at[idxb[b]], gb[b], gsem[b])
        pltpu.async_copy(ew_hbm.at[ss, b], ewb[b], wsem[b])
        pltpu.async_copy(sb[b], acc.at[dstb[b]], ssem[b], add=True)

    def _round(i, _):
        for b in range(DEPTH):
            c = i * DEPTH + b
            pltpu.make_async_copy(xs_hbm.at[idxb[b]], gb[b], gsem[b]).wait()
            _mk_src(c + DEPTH, idxb[b])
            pltpu.make_async_copy(ew_hbm.at[ss, c], ewb[b], wsem[b]).wait()
            pltpu.make_async_copy(sb[b], acc.at[dstb[b]], ssem[b]).wait()
            _mk_dst(c, dstb[b])
            _scale(gb[b], sb[b], ewb[b])
            pltpu.async_copy(xs_hbm.at[idxb[b]], gb[b], gsem[b])
            pltpu.async_copy(ew_hbm.at[ss, c + DEPTH], ewb[b], wsem[b])
            pltpu.async_copy(sb[b], acc.at[dstb[b]], ssem[b], add=True)
        return 0

    lax.fori_loop(0, NPROC // DEPTH, _round, 0)
    for b in range(DEPTH):
        pltpu.make_async_copy(xs_hbm.at[idxb[b]], gb[b], gsem[b]).wait()
        pltpu.make_async_copy(ew_hbm.at[ss, 0], ewb[b], wsem[b]).wait()
        pltpu.make_async_copy(sb[b], acc.at[dstb[b]], ssem[b]).wait()
    plsc.subcore_barrier()

    pltpu.sync_copy(acc.at[pl.ds(ss * RPT, RPT)],
                    out_hbm.at[cc, pl.ds(ss * RPT, RPT)])

    @pl.when(ss == NS - 1)
    def _write_tail():
        pltpu.sync_copy(acc.at[pl.ds(NS * RPT, RTAIL)],
                        out_hbm.at[cc, pl.ds(NS * RPT, RTAIL)])


_BM = 1000


def _mm_body(p_ref, w_ref, b_ref, o_ref):
    o_ref[...] = (
        jnp.dot(p_ref[0], w_ref[0], preferred_element_type=jnp.float32)
        + jnp.dot(p_ref[1], w_ref[1], preferred_element_type=jnp.float32)
        + b_ref[...]
    )


def _finish(partials, W2, b2):
    return pl.pallas_call(
        _mm_body,
        grid=(N // _BM,),
        in_specs=[
            pl.BlockSpec((NC, _BM, DH), lambda i: (0, i, 0)),
            pl.BlockSpec((NC, DH, D), lambda i: (0, 0, 0)),
            pl.BlockSpec((1, D), lambda i: (0, 0)),
        ],
        out_specs=pl.BlockSpec((_BM, D), lambda i: (i, 0)),
        out_shape=jax.ShapeDtypeStruct((N, D), jnp.float32),
    )(partials, W2, b2)


def kernel(X, edge_index, edge_weight, W, b):
    src = edge_index[0].astype(jnp.int32)
    dst = edge_index[1].astype(jnp.int32)
    npad = NPADC - NCHUNK
    sd = jnp.pad((src | (dst << 15)).reshape(NS, NCHUNK, CH),
                 ((0, 0), (0, npad), (0, 0)))
    ew = jnp.pad(edge_weight.reshape(NS, NCHUNK, CH),
                 ((0, 0), (0, npad), (0, 0)))
    xh = X.reshape(N, NC, DH).transpose(1, 0, 2)
    xbf = xh[:, :, _PERM].astype(jnp.bfloat16)
    xs = lax.bitcast_convert_type(
        xbf.reshape(NC, N, DP, 2), jnp.int32).reshape(NC * N, DP)
    partials = _aggregate(xs, sd, ew)
    w2 = jnp.stack([W[:DH], W[DH:]])
    return _finish(partials, w2, b.reshape(1, D))


# speedup vs baseline: 1.0683x; 1.0683x over previous
import functools

import jax
import jax.numpy as jnp
import numpy as np
from jax import lax
from jax.experimental import pallas as pl
from jax.experimental.pallas import tpu as pltpu
from jax.experimental.pallas import tpu_sc as plsc

N = 10000
E = 320000
D = 128
DH = D // 2
DP = DH // 2
NC = 2
NS = 16
EPT = E // NS
CH = 80
NCHUNK = EPT // CH
DEPTH = 4
NPROC = 252
NPADC = NPROC + DEPTH
RPT = 624
RTAIL = N - NS * RPT

_PERM = np.concatenate([
    np.stack([np.arange(0, 16), np.arange(32, 48)], 1).ravel(),
    np.stack([np.arange(16, 32), np.arange(48, 64)], 1).ravel(),
])

_mesh = plsc.VectorSubcoreMesh(core_axis_name="c", subcore_axis_name="s")


@functools.partial(
    pl.kernel,
    mesh=_mesh,
    compiler_params=pltpu.CompilerParams(
        use_tc_tiling_on_sc=False, needs_layout_passes=False),
    out_type=jax.ShapeDtypeStruct((NC, N, DH), jnp.float32),
    scratch_types=(
        [pltpu.VMEM((NPADC, CH), jnp.int32)]
        + [pltpu.VMEM((CH,), jnp.int32) for _ in range(DEPTH)]
        + [pltpu.VMEM((CH,), jnp.int32) for _ in range(DEPTH)]
        + [pltpu.VMEM((CH, DP), jnp.int32) for _ in range(DEPTH)]
        + [pltpu.VMEM((CH, DH), jnp.float32) for _ in range(DEPTH)]
        + [pltpu.VMEM((NPROC, CH), jnp.float32)]
        + [pltpu.VMEM_SHARED((N, DH), jnp.float32)]
        + [pltpu.SemaphoreType.DMA] * (2 * DEPTH)
    ),
)
def _aggregate(xs_hbm, sd_hbm, ew_hbm, out_hbm, *refs):
    sd_v = refs[0]
    idxb = refs[1:1 + DEPTH]
    dstb = refs[1 + DEPTH:1 + 2 * DEPTH]
    gb = refs[1 + 2 * DEPTH:1 + 3 * DEPTH]
    sb = refs[1 + 3 * DEPTH:1 + 4 * DEPTH]
    ew_v = refs[1 + 4 * DEPTH]
    acc = refs[2 + 4 * DEPTH]
    gsem = refs[3 + 4 * DEPTH:3 + 5 * DEPTH]
    ssem = refs[3 + 5 * DEPTH:3 + 6 * DEPTH]

    cc = lax.axis_index("c")
    ss = lax.axis_index("s")
    ccn = cc * N

    pltpu.sync_copy(sd_hbm.at[ss], sd_v)
    pltpu.sync_copy(ew_hbm.at[ss], ew_v)

    lomask = jnp.int32((1 << 15) - 1)

    def _mk_src(c, buf):
        for j in range(CH // 16):
            sl = pl.ds(j * 16, 16)
            buf[sl] = (sd_v[c, sl] & lomask) + ccn

    def _mk_dst(c, buf):
        for j in range(CH // 16):
            sl = pl.ds(j * 16, 16)
            buf[sl] = sd_v[c, sl] >> 15

    def _zrow(i, _):
        for b in range(DEPTH):
            for j in range(DH // 16):
                sb[b][i, pl.ds(j * 16, 16)] = jnp.zeros((16,), jnp.float32)
        return 0
    lax.fori_loop(0, CH, _zrow, 0)
    for b in range(DEPTH):
        for j in range(CH // 16):
            dstb[b][pl.ds(j * 16, 16)] = jnp.zeros((16,), jnp.int32)

    for k in range(7):
        pltpu.sync_copy(sb[0], acc.at[pl.ds(ss * RPT + k * CH, CH)])
    pltpu.sync_copy(sb[0].at[pl.ds(0, RPT - 7 * CH)],
                    acc.at[pl.ds(ss * RPT + 7 * CH, RPT - 7 * CH)])

    @pl.when(ss == NS - 1)
    def _zero_tail():
        pltpu.sync_copy(sb[0].at[pl.ds(0, RTAIL)],
                        acc.at[pl.ds(NS * RPT, RTAIL)])

    plsc.subcore_barrier()

    himask = jnp.int32(-65536)

    def _scale(gbuf, sbuf, ci):
        def _grp(g, _):
            wvec = ew_v[ci, pl.ds(g * 16, 16)]
            for l in range(16):
                e = g * 16 + l
                w = wvec[l]
                for h in range(2):
                    v = gbuf[e, pl.ds(h * 16, 16)]
                    lo = plsc.bitcast(v << 16, jnp.float32)
                    hi = plsc.bitcast(v & himask, jnp.float32)
                    sbuf[e, pl.ds(h * 16, 16)] = lo * w
                    sbuf[e, pl.ds(h * 16 + 32, 16)] = hi * w
            return 0
        lax.fori_loop(0, CH // 16, _grp, 0)

    for b in range(DEPTH):
        _mk_src(b, idxb[b])
        pltpu.async_copy(xs_hbm.at[idxb[b]], gb[b], gsem[b])
        pltpu.async_copy(sb[b], acc.at[dstb[b]], ssem[b], add=True)

    def _round(i, _):
        for b in range(DEPTH):
            c = i * DEPTH + b
            pltpu.make_async_copy(xs_hbm.at[idxb[b]], gb[b], gsem[b]).wait()
            _mk_src(c + DEPTH, idxb[b])
            pltpu.make_async_copy(sb[b], acc.at[dstb[b]], ssem[b]).wait()
            _mk_dst(c, dstb[b])
            _scale(gb[b], sb[b], c)
            pltpu.async_copy(xs_hbm.at[idxb[b]], gb[b], gsem[b])
            pltpu.async_copy(sb[b], acc.at[dstb[b]], ssem[b], add=True)
        return 0

    lax.fori_loop(0, NPROC // DEPTH, _round, 0)
    for b in range(DEPTH):
        pltpu.make_async_copy(xs_hbm.at[idxb[b]], gb[b], gsem[b]).wait()
        pltpu.make_async_copy(sb[b], acc.at[dstb[b]], ssem[b]).wait()
    plsc.subcore_barrier()

    pltpu.sync_copy(acc.at[pl.ds(ss * RPT, RPT)],
                    out_hbm.at[cc, pl.ds(ss * RPT, RPT)])

    @pl.when(ss == NS - 1)
    def _write_tail():
        pltpu.sync_copy(acc.at[pl.ds(NS * RPT, RTAIL)],
                        out_hbm.at[cc, pl.ds(NS * RPT, RTAIL)])


_BM = 1000


def _mm_body(p_ref, w_ref, b_ref, o_ref):
    o_ref[...] = (
        jnp.dot(p_ref[0], w_ref[0], preferred_element_type=jnp.float32)
        + jnp.dot(p_ref[1], w_ref[1], preferred_element_type=jnp.float32)
        + b_ref[...]
    )


def _finish(partials, W2, b2):
    return pl.pallas_call(
        _mm_body,
        grid=(N // _BM,),
        in_specs=[
            pl.BlockSpec((NC, _BM, DH), lambda i: (0, i, 0)),
            pl.BlockSpec((NC, DH, D), lambda i: (0, 0, 0)),
            pl.BlockSpec((1, D), lambda i: (0, 0)),
        ],
        out_specs=pl.BlockSpec((_BM, D), lambda i: (i, 0)),
        out_shape=jax.ShapeDtypeStruct((N, D), jnp.float32),
    )(partials, W2, b2)


def kernel(X, edge_index, edge_weight, W, b):
    src = edge_index[0].astype(jnp.int32)
    dst = edge_index[1].astype(jnp.int32)
    sd = jnp.pad((src | (dst << 15)).reshape(NS, NCHUNK, CH),
                 ((0, 0), (0, NPADC - NCHUNK), (0, 0)))
    ew = jnp.pad(edge_weight.reshape(NS, NCHUNK, CH),
                 ((0, 0), (0, NPROC - NCHUNK), (0, 0)))
    xh = X.reshape(N, NC, DH).transpose(1, 0, 2)
    xbf = xh[:, :, _PERM].astype(jnp.bfloat16)
    xs = lax.bitcast_convert_type(
        xbf.reshape(NC, N, DP, 2), jnp.int32).reshape(NC * N, DP)
    partials = _aggregate(xs, sd, ew)
    w2 = jnp.stack([W[:DH], W[DH:]])
    return _finish(partials, w2, b.reshape(1, D))


# speedup vs baseline: 1.2336x; 1.1548x over previous
import functools

import jax
import jax.numpy as jnp
import numpy as np
from jax import lax
from jax.experimental import pallas as pl
from jax.experimental.pallas import tpu as pltpu
from jax.experimental.pallas import tpu_sc as plsc

N = 10000
E = 320000
D = 128
DP = D // 2
NC = 2
NS = 16
NW = NC * NS
EPT = E // NW
CH = 80
NCHUNK = EPT // CH
DEPTH = 2
NPROC = 126
NPADC = NPROC + DEPTH
RPT = 624
RTAIL = N - NS * RPT

_PERM = np.stack([np.arange(0, 64), np.arange(64, 128)], 1).ravel()

_mesh = plsc.VectorSubcoreMesh(core_axis_name="c", subcore_axis_name="s")


@functools.partial(
    pl.kernel,
    mesh=_mesh,
    compiler_params=pltpu.CompilerParams(
        use_tc_tiling_on_sc=False, needs_layout_passes=False),
    out_type=jax.ShapeDtypeStruct((NC, N, D), jnp.bfloat16),
    scratch_types=(
        [pltpu.VMEM((NPADC, CH), jnp.int32)]
        + [pltpu.VMEM((CH,), jnp.int32) for _ in range(DEPTH)]
        + [pltpu.VMEM((CH,), jnp.int32) for _ in range(DEPTH)]
        + [pltpu.VMEM((CH, DP), jnp.int32) for _ in range(DEPTH)]
        + [pltpu.VMEM((CH, D), jnp.bfloat16) for _ in range(DEPTH)]
        + [pltpu.VMEM((NPROC, CH), jnp.float32)]
        + [pltpu.VMEM_SHARED((N, D), jnp.bfloat16)]
        + [pltpu.SemaphoreType.DMA] * (2 * DEPTH)
    ),
)
def _aggregate(xs_hbm, sd_hbm, ew_hbm, out_hbm, *refs):
    sd_v = refs[0]
    idxb = refs[1:1 + DEPTH]
    dstb = refs[1 + DEPTH:1 + 2 * DEPTH]
    gb = refs[1 + 2 * DEPTH:1 + 3 * DEPTH]
    sb = refs[1 + 3 * DEPTH:1 + 4 * DEPTH]
    ew_v = refs[1 + 4 * DEPTH]
    acc = refs[2 + 4 * DEPTH]
    gsem = refs[3 + 4 * DEPTH:3 + 5 * DEPTH]
    ssem = refs[3 + 5 * DEPTH:3 + 6 * DEPTH]

    cc = lax.axis_index("c")
    ss = lax.axis_index("s")
    t = cc * NS + ss

    pltpu.sync_copy(sd_hbm.at[t], sd_v)
    pltpu.sync_copy(ew_hbm.at[t], ew_v)

    lomask = jnp.int32((1 << 15) - 1)

    def _mk_src(c, buf):
        for j in range(CH // 16):
            sl = pl.ds(j * 16, 16)
            buf[sl] = sd_v[c, sl] & lomask

    def _mk_dst(c, buf):
        for j in range(CH // 16):
            sl = pl.ds(j * 16, 16)
            buf[sl] = sd_v[c, sl] >> 15

    zb = jnp.zeros((32,), jnp.bfloat16)

    def _zrow(i, _):
        for b in range(DEPTH):
            for j in range(D // 32):
                sb[b][i, pl.ds(j * 32, 32)] = zb
        return 0
    lax.fori_loop(0, CH, _zrow, 0)
    for b in range(DEPTH):
        for j in range(CH // 16):
            dstb[b][pl.ds(j * 16, 16)] = jnp.zeros((16,), jnp.int32)

    for k in range(7):
        pltpu.sync_copy(sb[0], acc.at[pl.ds(ss * RPT + k * CH, CH)])
    pltpu.sync_copy(sb[0].at[pl.ds(0, RPT - 7 * CH)],
                    acc.at[pl.ds(ss * RPT + 7 * CH, RPT - 7 * CH)])

    @pl.when(ss == NS - 1)
    def _zero_tail():
        pltpu.sync_copy(sb[0].at[pl.ds(0, RTAIL)],
                        acc.at[pl.ds(NS * RPT, RTAIL)])

    plsc.subcore_barrier()

    himask = jnp.int32(-65536)

    def _scale(gbuf, sbuf, ci):
        def _grp(g, _):
            wvec = ew_v[ci, pl.ds(g * 16, 16)]
            for l in range(16):
                e = g * 16 + l
                w = wvec[l]
                for h in range(DP // 16):
                    v = gbuf[e, pl.ds(h * 16, 16)]
                    lo = plsc.bitcast(v << 16, jnp.float32)
                    hi = plsc.bitcast(v & himask, jnp.float32)
                    sbuf[e, pl.ds(h * 32, 32)] = plsc.pack(
                        lo * w, hi * w, format=plsc.PackFormat.INTERLEAVED)
            return 0
        lax.fori_loop(0, CH // 16, _grp, 0)

    for b in range(DEPTH):
        _mk_src(b, idxb[b])
        pltpu.async_copy(xs_hbm.at[idxb[b]], gb[b], gsem[b])
        pltpu.async_copy(sb[b], acc.at[dstb[b]], ssem[b], add=True)

    def _round(i, _):
        for b in range(DEPTH):
            c = i * DEPTH + b
            pltpu.make_async_copy(xs_hbm.at[idxb[b]], gb[b], gsem[b]).wait()
            _mk_src(c + DEPTH, idxb[b])
            pltpu.make_async_copy(sb[b], acc.at[dstb[b]], ssem[b]).wait()
            _mk_dst(c, dstb[b])
            _scale(gb[b], sb[b], c)
            pltpu.async_copy(xs_hbm.at[idxb[b]], gb[b], gsem[b])
            pltpu.async_copy(sb[b], acc.at[dstb[b]], ssem[b], add=True)
        return 0

    lax.fori_loop(0, NPROC // DEPTH, _round, 0)
    for b in range(DEPTH):
        pltpu.make_async_copy(xs_hbm.at[idxb[b]], gb[b], gsem[b]).wait()
        pltpu.make_async_copy(sb[b], acc.at[dstb[b]], ssem[b]).wait()
    plsc.subcore_barrier()

    pltpu.sync_copy(acc.at[pl.ds(ss * RPT, RPT)],
                    out_hbm.at[cc, pl.ds(ss * RPT, RPT)])

    @pl.when(ss == NS - 1)
    def _write_tail():
        pltpu.sync_copy(acc.at[pl.ds(NS * RPT, RTAIL)],
                        out_hbm.at[cc, pl.ds(NS * RPT, RTAIL)])


_BM = 1000


def _mm_body(p_ref, w_ref, b_ref, o_ref):
    o_ref[...] = (
        jnp.dot(p_ref[0], w_ref[...], preferred_element_type=jnp.float32)
        + jnp.dot(p_ref[1], w_ref[...], preferred_element_type=jnp.float32)
        + b_ref[...]
    )


def _finish(partials, Wp, b2):
    return pl.pallas_call(
        _mm_body,
        grid=(N // _BM,),
        in_specs=[
            pl.BlockSpec((NC, _BM, D), lambda i: (0, i, 0)),
            pl.BlockSpec((D, D), lambda i: (0, 0)),
            pl.BlockSpec((1, D), lambda i: (0, 0)),
        ],
        out_specs=pl.BlockSpec((_BM, D), lambda i: (i, 0)),
        out_shape=jax.ShapeDtypeStruct((N, D), jnp.float32),
    )(partials, Wp, b2)


def kernel(X, edge_index, edge_weight, W, b):
    src = edge_index[0].astype(jnp.int32)
    dst = edge_index[1].astype(jnp.int32)
    sd = jnp.pad((src | (dst << 15)).reshape(NW, NCHUNK, CH),
                 ((0, 0), (0, NPADC - NCHUNK), (0, 0)))
    ew = jnp.pad(edge_weight.reshape(NW, NCHUNK, CH),
                 ((0, 0), (0, NPROC - NCHUNK), (0, 0)))
    xbf = X[:, _PERM].astype(jnp.bfloat16)
    xs = lax.bitcast_convert_type(xbf.reshape(N, DP, 2), jnp.int32)
    partials = _aggregate(xs, sd, ew)
    wp = W[_PERM]
    return _finish(partials, wp, b.reshape(1, D))
